# baseline (speedup 1.0000x reference)
import functools

import jax
import jax.numpy as jnp
from jax import lax
from jax.experimental import pallas as pl
from jax.experimental.pallas import tpu as pltpu
from jax.experimental.pallas import tpu_sc as plsc

BATCH = 16384
NFACT = 32
NROWS = 1000000
CHUNK = 128
PACK = 128 // NFACT
WIN = 32768
NWIN = (NROWS + WIN - 1) // WIN
PACKED_ROWS = NWIN * (WIN // PACK)
W_TC = 20
W_SC = NWIN - 1 - W_TC
SC_BASE_I32 = W_TC * (WIN // PACK // 2)
SC_I32_ROWS = W_SC * (WIN // PACK // 2)


def _relayout_body(u_ref, m_ref, e_ref, uo_ref, mo_ref):
    q = WIN // PACK
    e = e_ref[...]
    for src, dst in ((u_ref, uo_ref), (m_ref, mo_ref)):
        x = src[...]
        acc = jnp.zeros((q, 128), jnp.float32)
        for a in range(PACK):
            acc = acc + jnp.dot(
                x[:, a * q:(a + 1) * q].astype(jnp.bfloat16).T,
                e[:, a * 128:(a + 1) * 128].astype(jnp.bfloat16),
                preferred_element_type=jnp.float32)
        dst[...] = pltpu.bitcast(acc.astype(jnp.bfloat16), jnp.int32)


def _tc_relayout(u_tt, m_tt, e_sel):
    grid = (W_TC + 1,)
    return pl.pallas_call(
        _relayout_body,
        grid=grid,
        compiler_params=pltpu.CompilerParams(
            fuse_transposed_lhs_in_matmul=True),
        in_specs=[
            pl.BlockSpec((NFACT, WIN),
                         lambda i: (0, jnp.where(i < W_TC, i, NWIN - 1))),
            pl.BlockSpec((NFACT, WIN),
                         lambda i: (0, jnp.where(i < W_TC, i, NWIN - 1))),
            pl.BlockSpec((NFACT, PACK * 128), lambda i: (0, 0)),
        ],
        out_specs=[
            pl.BlockSpec((WIN // PACK // 2, 128),
                         lambda i: (jnp.where(i < W_TC, i, NWIN - 1), 0)),
            pl.BlockSpec((WIN // PACK // 2, 128),
                         lambda i: (jnp.where(i < W_TC, i, NWIN - 1), 0)),
        ],
        out_shape=[
            jax.ShapeDtypeStruct((PACKED_ROWS // 2, 128), jnp.int32),
            jax.ShapeDtypeStruct((PACKED_ROWS // 2, 128), jnp.int32),
        ],
    )(u_tt, m_tt, e_sel)


def _make_sc_relayout(num_cores, num_subcores):
    nw = num_cores * num_subcores
    n_units = W_SC * (WIN // PACK // 128)
    u_per_w = n_units // nw
    mesh = plsc.VectorSubcoreMesh(core_axis_name="c", subcore_axis_name="s")

    @functools.partial(
        pl.kernel,
        mesh=mesh,
        compiler_params=pltpu.CompilerParams(needs_layout_passes=False),
        out_type=[
            jax.ShapeDtypeStruct((SC_I32_ROWS, 128), jnp.int32),
            jax.ShapeDtypeStruct((SC_I32_ROWS, 128), jnp.int32),
        ],
        scratch_types=[
            pltpu.VMEM((2, PACK * NFACT, 128), jnp.float32),
            pltpu.VMEM((2, PACK * NFACT, 128), jnp.float32),
            pltpu.VMEM((64, 128), jnp.int32),
            pltpu.VMEM((64, 128), jnp.int32),
            pltpu.SemaphoreType.DMA,
            pltpu.SemaphoreType.DMA,
        ],
    )
    def sc_relayout(ut_hbm, mt_hbm, uo_hbm, mo_hbm,
                    ubuf, mbuf, uoutv, moutv, su, sm):
        wid = lax.axis_index("s") * num_cores + lax.axis_index("c")

        def start(unit, slot):
            g = wid * u_per_w + unit
            col0 = (W_TC + (g >> 6)) * WIN + (g & 63) * 128
            col0 = pl.multiple_of(col0, 128)
            cps = []
            for a in range(PACK):
                c = pl.multiple_of(col0 + a * (WIN // PACK), 128)
                cps.append(pltpu.async_copy(
                    ut_hbm.at[:, pl.ds(c, 128)],
                    ubuf.at[slot, pl.ds(a * NFACT, NFACT)], su))
                cps.append(pltpu.async_copy(
                    mt_hbm.at[:, pl.ds(c, 128)],
                    mbuf.at[slot, pl.ds(a * NFACT, NFACT)], sm))
            return cps

        def transpose_pack(buf, slot, outv):
            for a in range(PACK):
                def kbody(k, carry):
                    for gp in range(4):
                        v0 = buf[slot, a * NFACT + k, pl.ds(gp * 32, 16)]
                        v1 = buf[slot, a * NFACT + k, pl.ds(gp * 32 + 16, 16)]
                        pk = plsc.bitcast(
                            plsc.pack(v0, v1,
                                      format=plsc.PackFormat.INTERLEAVED),
                            jnp.int32)
                        rows = gp * 16 + lax.iota(jnp.int32, 16)
                        cols = jnp.full((16,), a * NFACT, jnp.int32) + k
                        plsc.store_scatter(outv, [rows, cols], pk)
                    return carry
                lax.fori_loop(0, NFACT, kbody, 0)

        def drain(cps):
            for c in cps:
                c.wait()

        def pair_body(i, carry):
            u0 = i * 2
            u1 = i * 2 + 1
            cps0 = start(u0, 0)
            cps1 = start(u1, 1)
            drain(cps0)
            transpose_pack(ubuf, 0, uoutv)
            transpose_pack(mbuf, 0, moutv)
            row0 = (wid * u_per_w + u0) * 64
            pltpu.sync_copy(uoutv, uo_hbm.at[pl.ds(row0, 64)])
            pltpu.sync_copy(moutv, mo_hbm.at[pl.ds(row0, 64)])
            drain(cps1)
            transpose_pack(ubuf, 1, uoutv)
            transpose_pack(mbuf, 1, moutv)
            row1 = (wid * u_per_w + u1) * 64
            pltpu.sync_copy(uoutv, uo_hbm.at[pl.ds(row1, 64)])
            pltpu.sync_copy(moutv, mo_hbm.at[pl.ds(row1, 64)])
            return carry

        lax.fori_loop(0, u_per_w // 2, pair_body, 0)

    return sc_relayout


def _make_sc_gather(num_cores, num_subcores):
    nw = num_cores * num_subcores
    b_per_w = BATCH // nw
    n_chunks = b_per_w // 64
    out_rows_w = b_per_w // PACK
    mesh = plsc.VectorSubcoreMesh(core_axis_name="c", subcore_axis_name="s")

    @functools.partial(
        pl.kernel,
        mesh=mesh,
        compiler_params=pltpu.CompilerParams(needs_layout_passes=False),
        out_type=[
            jax.ShapeDtypeStruct((BATCH // PACK, 128), jnp.float32),
            jax.ShapeDtypeStruct((BATCH // PACK, 128), jnp.float32),
        ],
        scratch_types=[
            pltpu.VMEM((n_chunks // 2, 128), jnp.int32),
            pltpu.VMEM((n_chunks // 2, 128), jnp.int32),
            pltpu.VMEM((n_chunks // 2, 128), jnp.int32),
            pltpu.VMEM((n_chunks // 2, 128), jnp.int32),
            pltpu.VMEM((n_chunks // 2, 128), jnp.int32),
            pltpu.VMEM((n_chunks // 2, 128), jnp.int32),
            pltpu.VMEM((2, 64, 128), jnp.int32),
            pltpu.VMEM((2, 64, 128), jnp.int32),
            pltpu.VMEM((2, 64, 128), jnp.int32),
            pltpu.VMEM((2, 64, 128), jnp.int32),
            pltpu.VMEM((out_rows_w, 128), jnp.float32),
            pltpu.VMEM((out_rows_w, 128), jnp.float32),
            pltpu.SemaphoreType.DMA,
            pltpu.SemaphoreType.DMA,
            pltpu.SemaphoreType.DMA,
            pltpu.SemaphoreType.DMA,
        ],
    )
    def sc_gather(users_hbm, movies_hbm, ut_hbm, mt_hbm, usc_hbm, msc_hbm,
                  uo_hbm, mo_hbm,
                  uraw, mraw, uprow, mprow, upsc, mpsc,
                  ubuf, mbuf, ubuf2, mbuf2, uout, mout,
                  su0, su1, sm0, sm1):
        sems_u = (su0, su1)
        sems_m = (sm0, sm1)
        wid = lax.axis_index("s") * num_cores + lax.axis_index("c")
        pltpu.sync_copy(users_hbm.at[wid], uraw)
        pltpu.sync_copy(movies_hbm.at[wid], mraw)
        for j in range(n_chunks // 2):
            for t in range(CHUNK // 16):
                s = pl.ds(t * 16, 16)
                for raw, prow, psc in ((uraw, uprow, upsc),
                                       (mraw, mprow, mpsc)):
                    r = raw[j, s]
                    p = ((lax.shift_right_logical(r, 15) << 13) + (r & 8191))
                    prow[j, s] = lax.shift_right_logical(p, 1)
                    wv = lax.shift_right_logical(r, 15)
                    rr = r & 8191
                    pl_sc = (((wv - W_TC) << 12)
                             + ((lax.shift_right_logical(rr, 7)) << 6)
                             + ((lax.shift_right_logical(rr, 5) & 3) << 4)
                             + (rr & 15))
                    psc[j, s] = jnp.minimum(
                        jnp.maximum(pl_sc, 0), SC_I32_ROWS - 1)

        def start(j):
            slot = j % 2
            j2, off = j >> 1, (j & 1) * 64
            cps = (
                pltpu.async_copy(ut_hbm.at[uprow.at[j2, pl.ds(off, 64)]],
                                 ubuf.at[slot], sems_u[slot]),
                pltpu.async_copy(usc_hbm.at[upsc.at[j2, pl.ds(off, 64)]],
                                 ubuf2.at[slot], sems_u[slot]),
                pltpu.async_copy(mt_hbm.at[mprow.at[j2, pl.ds(off, 64)]],
                                 mbuf.at[slot], sems_m[slot]),
                pltpu.async_copy(msc_hbm.at[mpsc.at[j2, pl.ds(off, 64)]],
                                 mbuf2.at[slot], sems_m[slot]),
            )
            return cps

        def extract(j, raw, buf, buf2, out):
            slot = j % 2
            j2, off = j >> 1, (j & 1) * 64

            def tbody(t, carry):
                iv = raw[j2, pl.ds(off + t * 16, 16)]
                ov = (lax.shift_right_logical(iv, 13) & 3) << 5
                wv = lax.shift_right_logical(iv, 15)
                s2v = jnp.where((wv >= W_TC) & (wv < NWIN - 1), 1, 0)
                sv = jnp.where(s2v == 1,
                               lax.shift_right_logical(iv, 4) & 1, iv & 1)
                for l in range(16):
                    o = ov[l]
                    sel = sv[l]
                    s2 = s2v[l]
                    r = t * 16 + l
                    orow = j * 16 + t * 4 + (l >> 2)
                    ocol = (l & 3) * NFACT
                    for h in range(2):
                        w1 = buf[slot, r, pl.ds(o + h * 16, 16)]
                        w2 = buf2[slot, r, pl.ds(o + h * 16, 16)]
                        w = jnp.where(s2 == 0, w1, w2)
                        lo = w << 16
                        hi = w & jnp.int32(-65536)
                        bits = jnp.where(sel == 0, lo, hi)
                        out[orow, pl.ds(ocol + h * 16, 16)] = plsc.bitcast(
                            bits, jnp.float32)
                return carry

            lax.fori_loop(0, 4, tbody, 0)

        pend = start(0)
        for j in range(n_chunks):
            cu1, cu2, cm1, cm2 = pend
            if j + 1 < n_chunks:
                pend = start(j + 1)
            cu1.wait()
            cu2.wait()
            extract(j, uraw, ubuf, ubuf2, uout)
            cm1.wait()
            cm2.wait()
            extract(j, mraw, mbuf, mbuf2, mout)

        base = wid * out_rows_w
        pltpu.sync_copy(uout, uo_hbm.at[pl.ds(base, out_rows_w)])
        pltpu.sync_copy(mout, mo_hbm.at[pl.ds(base, out_rows_w)])

    return sc_gather


def _mlp_body(u_ref, m_ref, a_ref, b_ref, c_ref, b1_ref, w2_ref, b2_ref, o_ref):
    u = u_ref[...]
    m = m_ref[...]
    e = u * m
    h = (jnp.dot(e, a_ref[...], preferred_element_type=jnp.float32)
         + jnp.dot(u, b_ref[...], preferred_element_type=jnp.float32)
         + jnp.dot(m, c_ref[...], preferred_element_type=jnp.float32)
         + b1_ref[...])
    h = jnp.maximum(h, 0.0)
    o = jnp.dot(h, w2_ref[...], preferred_element_type=jnp.float32) + b2_ref[...]
    o_ref[...] = jax.nn.sigmoid(o)


def _tc_mlp(u128, m128, a_bd, b_bd, c_bd, b1t, w2_bd, b2t):
    rows = 512
    grid = ((BATCH // PACK) // rows,)
    wspec = lambda shape: pl.BlockSpec(shape, lambda i: (0, 0))
    return pl.pallas_call(
        _mlp_body,
        grid=grid,
        in_specs=[
            pl.BlockSpec((rows, 128), lambda i: (i, 0)),
            pl.BlockSpec((rows, 128), lambda i: (i, 0)),
            wspec((128, PACK * 8)),
            wspec((128, PACK * 8)),
            wspec((128, PACK * 8)),
            wspec((1, PACK * 8)),
            wspec((PACK * 8, PACK)),
            wspec((1, PACK)),
        ],
        out_specs=pl.BlockSpec((rows, PACK), lambda i: (i, 0)),
        out_shape=jax.ShapeDtypeStruct((BATCH // PACK, PACK), jnp.float32),
    )(u128, m128, a_bd, b_bd, c_bd, b1t, w2_bd, b2t)


def kernel(users, movies, user_table, movie_table, W1, b1, W2, b2):
    info = plsc.get_sparse_core_info()
    nc, ns = info.num_cores, info.num_subcores
    nw = nc * ns
    b_per_w = BATCH // nw
    n_chunks = b_per_w // CHUNK
    i32eye = jnp.eye(NFACT, dtype=jnp.float32)
    e_sel = jnp.zeros((NFACT, PACK * 128), jnp.float32)
    for a in range(PACK):
        s = a * 128 + a * NFACT
        e_sel = e_sel.at[:, s:s + NFACT].set(i32eye)
    ut_c, mt_c = _tc_relayout(user_table.T, movie_table.T, e_sel)
    sc_relayout = _make_sc_relayout(nc, ns)
    usc, msc = sc_relayout(user_table.T, movie_table.T)
    sc_gather = _make_sc_gather(nc, ns)
    users_r = users.astype(jnp.int32).reshape(nw, n_chunks, CHUNK)
    movies_r = movies.astype(jnp.int32).reshape(nw, n_chunks, CHUNK)
    u128, m128 = sc_gather(users_r, movies_r, ut_c, mt_c, usc, msc)

    eye = jnp.eye(PACK, dtype=jnp.float32)
    a_bd = jnp.kron(eye, W1[0:NFACT])
    b_bd = jnp.kron(eye, W1[NFACT:2 * NFACT])
    c_bd = jnp.kron(eye, W1[2 * NFACT:3 * NFACT])
    w2_bd = jnp.kron(eye, W2)
    b1t = jnp.tile(b1, PACK).reshape(1, PACK * 8)
    b2t = jnp.broadcast_to(b2.reshape(1, 1), (1, PACK))
    out = _tc_mlp(u128, m128, a_bd, b_bd, c_bd, b1t, w2_bd, b2t)
    return out.reshape(BATCH, 1)


# speedup vs baseline: 3.5962x; 3.5962x over previous
import functools

import jax
import jax.numpy as jnp
from jax import lax
from jax.experimental import pallas as pl
from jax.experimental.pallas import tpu as pltpu
from jax.experimental.pallas import tpu_sc as plsc

BATCH = 16384
NFACT = 32
NROWS = 1000000
CHUNK = 128
PACK = 128 // NFACT
WIN = 32768
NWIN = (NROWS + WIN - 1) // WIN
PACKED_ROWS = NWIN * (WIN // PACK)


def _relayout_body(u_ref, m_ref, e_ref, uo_ref, mo_ref):
    q = WIN // PACK
    e = e_ref[...]
    for src, dst in ((u_ref, uo_ref), (m_ref, mo_ref)):
        x = src[...]
        acc = jnp.zeros((q, 128), jnp.float32)
        for a in range(PACK):
            acc = acc + jnp.dot(
                x[:, a * q:(a + 1) * q].astype(jnp.bfloat16).T,
                e[:, a * 128:(a + 1) * 128].astype(jnp.bfloat16),
                preferred_element_type=jnp.float32)
        dst[...] = pltpu.bitcast(acc.astype(jnp.bfloat16), jnp.int32)


def _tc_relayout(u_tt, m_tt, e_sel):
    grid = (NWIN,)
    return pl.pallas_call(
        _relayout_body,
        grid=grid,
        compiler_params=pltpu.CompilerParams(
            fuse_transposed_lhs_in_matmul=True),
        in_specs=[
            pl.BlockSpec((NFACT, WIN), lambda i: (0, i)),
            pl.BlockSpec((NFACT, WIN), lambda i: (0, i)),
            pl.BlockSpec((NFACT, PACK * 128), lambda i: (0, 0)),
        ],
        out_specs=[
            pl.BlockSpec((WIN // PACK // 2, 128), lambda i: (i, 0)),
            pl.BlockSpec((WIN // PACK // 2, 128), lambda i: (i, 0)),
        ],
        out_shape=[
            jax.ShapeDtypeStruct((PACKED_ROWS // 2, 128), jnp.int32),
            jax.ShapeDtypeStruct((PACKED_ROWS // 2, 128), jnp.int32),
        ],
    )(u_tt, m_tt, e_sel)


def _make_sc_gather(num_cores, num_subcores):
    nw = num_cores * num_subcores
    b_per_w = BATCH // nw
    n_chunks = b_per_w // CHUNK
    out_rows_w = b_per_w // PACK
    mesh = plsc.VectorSubcoreMesh(core_axis_name="c", subcore_axis_name="s")

    @functools.partial(
        pl.kernel,
        mesh=mesh,
        compiler_params=pltpu.CompilerParams(needs_layout_passes=False),
        out_type=[
            jax.ShapeDtypeStruct((BATCH // PACK, 128), jnp.float32),
            jax.ShapeDtypeStruct((BATCH // PACK, 128), jnp.float32),
        ],
        scratch_types=[
            pltpu.VMEM((n_chunks, CHUNK), jnp.int32),
            pltpu.VMEM((n_chunks, CHUNK), jnp.int32),
            pltpu.VMEM((n_chunks, CHUNK), jnp.int32),
            pltpu.VMEM((n_chunks, CHUNK), jnp.int32),
            pltpu.VMEM((2, CHUNK, 128), jnp.int32),
            pltpu.VMEM((2, CHUNK, 128), jnp.int32),
            pltpu.VMEM((out_rows_w, 128), jnp.float32),
            pltpu.VMEM((out_rows_w, 128), jnp.float32),
            pltpu.SemaphoreType.DMA,
            pltpu.SemaphoreType.DMA,
            pltpu.SemaphoreType.DMA,
            pltpu.SemaphoreType.DMA,
        ],
    )
    def sc_gather(users_hbm, movies_hbm, ut_hbm, mt_hbm, uo_hbm, mo_hbm,
                  uraw, mraw, uprow, mprow, ubuf, mbuf, uout, mout,
                  su0, su1, sm0, sm1):
        sems_u = (su0, su1)
        sems_m = (sm0, sm1)
        wid = lax.axis_index("s") * num_cores + lax.axis_index("c")
        pltpu.sync_copy(users_hbm.at[wid], uraw)
        pltpu.sync_copy(movies_hbm.at[wid], mraw)
        for j in range(n_chunks):
            for t in range(CHUNK // 16):
                s = pl.ds(t * 16, 16)
                ru = uraw[j, s]
                rm = mraw[j, s]
                pu = ((lax.shift_right_logical(ru, 15) << 13) + (ru & 8191))
                pm = ((lax.shift_right_logical(rm, 15) << 13) + (rm & 8191))
                uprow[j, s] = lax.shift_right_logical(pu, 1)
                mprow[j, s] = lax.shift_right_logical(pm, 1)

        def start(j):
            slot = j % 2
            cu = pltpu.async_copy(ut_hbm.at[uprow.at[j]], ubuf.at[slot],
                                  sems_u[slot])
            cm = pltpu.async_copy(mt_hbm.at[mprow.at[j]], mbuf.at[slot],
                                  sems_m[slot])
            return cu, cm

        def extract(j, raw, buf, out):
            slot = j % 2

            def tbody(t, carry):
                iv = raw[j, pl.ds(t * 16, 16)]
                ov = (lax.shift_right_logical(iv, 13) & 3) << 5
                sv = iv & 1
                for l in range(16):
                    o = ov[l]
                    sel = sv[l]
                    r = t * 16 + l
                    orow = j * (CHUNK // PACK) + t * 4 + (l >> 2)
                    ocol = (l & 3) * NFACT
                    for h in range(2):
                        w = buf[slot, r, pl.ds(o + h * 16, 16)]
                        lo = w << 16
                        hi = w & jnp.int32(-65536)
                        bits = jnp.where(sel == 0, lo, hi)
                        out[orow, pl.ds(ocol + h * 16, 16)] = plsc.bitcast(
                            bits, jnp.float32)
                return carry

            lax.fori_loop(0, CHUNK // 16, tbody, 0)

        pend = start(0)
        for j in range(n_chunks):
            cu, cm = pend
            if j + 1 < n_chunks:
                pend = start(j + 1)
            cu.wait()
            extract(j, uraw, ubuf, uout)
            cm.wait()
            extract(j, mraw, mbuf, mout)

        base = wid * out_rows_w
        pltpu.sync_copy(uout, uo_hbm.at[pl.ds(base, out_rows_w)])
        pltpu.sync_copy(mout, mo_hbm.at[pl.ds(base, out_rows_w)])

    return sc_gather


def _mlp_body(u_ref, m_ref, a_ref, b_ref, c_ref, b1_ref, w2_ref, b2_ref, o_ref):
    u = u_ref[...]
    m = m_ref[...]
    e = u * m
    h = (jnp.dot(e, a_ref[...], preferred_element_type=jnp.float32)
         + jnp.dot(u, b_ref[...], preferred_element_type=jnp.float32)
         + jnp.dot(m, c_ref[...], preferred_element_type=jnp.float32)
         + b1_ref[...])
    h = jnp.maximum(h, 0.0)
    o = jnp.dot(h, w2_ref[...], preferred_element_type=jnp.float32) + b2_ref[...]
    o_ref[...] = jax.nn.sigmoid(o)


def _tc_mlp(u128, m128, a_bd, b_bd, c_bd, b1t, w2_bd, b2t):
    rows = 512
    grid = ((BATCH // PACK) // rows,)
    wspec = lambda shape: pl.BlockSpec(shape, lambda i: (0, 0))
    return pl.pallas_call(
        _mlp_body,
        grid=grid,
        in_specs=[
            pl.BlockSpec((rows, 128), lambda i: (i, 0)),
            pl.BlockSpec((rows, 128), lambda i: (i, 0)),
            wspec((128, PACK * 8)),
            wspec((128, PACK * 8)),
            wspec((128, PACK * 8)),
            wspec((1, PACK * 8)),
            wspec((PACK * 8, PACK)),
            wspec((1, PACK)),
        ],
        out_specs=pl.BlockSpec((rows, PACK), lambda i: (i, 0)),
        out_shape=jax.ShapeDtypeStruct((BATCH // PACK, PACK), jnp.float32),
    )(u128, m128, a_bd, b_bd, c_bd, b1t, w2_bd, b2t)


def kernel(users, movies, user_table, movie_table, W1, b1, W2, b2):
    info = plsc.get_sparse_core_info()
    nc, ns = info.num_cores, info.num_subcores
    nw = nc * ns
    b_per_w = BATCH // nw
    n_chunks = b_per_w // CHUNK
    i32eye = jnp.eye(NFACT, dtype=jnp.float32)
    e_sel = jnp.zeros((NFACT, PACK * 128), jnp.float32)
    for a in range(PACK):
        s = a * 128 + a * NFACT
        e_sel = e_sel.at[:, s:s + NFACT].set(i32eye)
    ut_c, mt_c = _tc_relayout(user_table.T, movie_table.T, e_sel)
    sc_gather = _make_sc_gather(nc, ns)
    users_r = users.astype(jnp.int32).reshape(nw, n_chunks, CHUNK)
    movies_r = movies.astype(jnp.int32).reshape(nw, n_chunks, CHUNK)
    u128, m128 = sc_gather(users_r, movies_r, ut_c, mt_c)

    eye = jnp.eye(PACK, dtype=jnp.float32)
    a_bd = jnp.kron(eye, W1[0:NFACT])
    b_bd = jnp.kron(eye, W1[NFACT:2 * NFACT])
    c_bd = jnp.kron(eye, W1[2 * NFACT:3 * NFACT])
    w2_bd = jnp.kron(eye, W2)
    b1t = jnp.tile(b1, PACK).reshape(1, PACK * 8)
    b2t = jnp.broadcast_to(b2.reshape(1, 1), (1, PACK))
    out = _tc_mlp(u128, m128, a_bd, b_bd, c_bd, b1t, w2_bd, b2t)
    return out.reshape(BATCH, 1)
